# baseline (speedup 1.0000x reference)
import jax
import jax.numpy as jnp
from jax.experimental import pallas as pl
from jax.experimental.pallas import tpu as pltpu

_B = 32
_C = 64
_P = 128
_L = 16
_COLS = _P
_ROWS = _B * _C * _L
_BLOCK_ROWS = 2048


def _mask_kernel(x_ref, moe_ref, meo_ref, oe_ref, eo_ref):
    x = x_ref[...]
    shape = x.shape
    row = jax.lax.broadcasted_iota(jnp.int32, shape, 0)
    col = jax.lax.broadcasted_iota(jnp.int32, shape, 1)
    oe = ((row // _L) ^ col) & 1
    eo = oe ^ 1
    oe_ref[...] = oe
    eo_ref[...] = eo
    zero = jnp.zeros_like(x)
    moe_ref[...] = jnp.where(oe == 1, zero, x)
    meo_ref[...] = jnp.where(oe == 0, zero, x)


def kernel(inputs):
    x2d = jnp.transpose(inputs, (0, 1, 3, 2)).reshape(_ROWS, _COLS)
    grid = (_ROWS // _BLOCK_ROWS,)
    spec = pl.BlockSpec((_BLOCK_ROWS, _COLS), lambda i: (i, 0))
    out = pl.pallas_call(
        _mask_kernel,
        grid=grid,
        in_specs=[spec],
        out_specs=[spec, spec, spec, spec],
        out_shape=[
            jax.ShapeDtypeStruct((_ROWS, _COLS), jnp.float32),
            jax.ShapeDtypeStruct((_ROWS, _COLS), jnp.float32),
            jax.ShapeDtypeStruct((_ROWS, _COLS), jnp.int32),
            jax.ShapeDtypeStruct((_ROWS, _COLS), jnp.int32),
        ],
        compiler_params=pltpu.CompilerParams(
            dimension_semantics=("parallel",),
        ),
    )(x2d)

    def _back(a):
        return jnp.transpose(
            a.reshape(_B, _C, _L, _P), (0, 1, 3, 2)
        )

    return tuple(_back(a) for a in out)


# speedup vs baseline: 1.0706x; 1.0706x over previous
import jax
import jax.numpy as jnp
from jax.experimental import pallas as pl
from jax.experimental.pallas import tpu as pltpu

_B = 32
_C = 64
_P = 128
_L = 16
_COLS = _P
_ROWS = _B * _C * _L
_BLOCK_ROWS = 4096


def _mask_kernel(x_ref, moe_ref, meo_ref, oe_ref, eo_ref):
    x = x_ref[...]
    shape = x.shape
    row = jax.lax.broadcasted_iota(jnp.int32, shape, 0)
    col = jax.lax.broadcasted_iota(jnp.int32, shape, 1)
    oe = ((row // _L) ^ col) & 1
    eo = oe ^ 1
    oe_ref[...] = oe
    eo_ref[...] = eo
    zero = jnp.zeros_like(x)
    moe_ref[...] = jnp.where(oe == 1, zero, x)
    meo_ref[...] = jnp.where(oe == 0, zero, x)


def kernel(inputs):
    x2d = jnp.transpose(inputs, (0, 1, 3, 2)).reshape(_ROWS, _COLS)
    grid = (_ROWS // _BLOCK_ROWS,)
    spec = pl.BlockSpec((_BLOCK_ROWS, _COLS), lambda i: (i, 0))
    out = pl.pallas_call(
        _mask_kernel,
        grid=grid,
        in_specs=[spec],
        out_specs=[spec, spec, spec, spec],
        out_shape=[
            jax.ShapeDtypeStruct((_ROWS, _COLS), jnp.float32),
            jax.ShapeDtypeStruct((_ROWS, _COLS), jnp.float32),
            jax.ShapeDtypeStruct((_ROWS, _COLS), jnp.int32),
            jax.ShapeDtypeStruct((_ROWS, _COLS), jnp.int32),
        ],
        compiler_params=pltpu.CompilerParams(
            dimension_semantics=("parallel",),
        ),
    )(x2d)

    def _back(a):
        return jnp.transpose(
            a.reshape(_B, _C, _L, _P), (0, 1, 3, 2)
        )

    return tuple(_back(a) for a in out)


# speedup vs baseline: 1.0943x; 1.0222x over previous
import jax
import jax.numpy as jnp
from jax.experimental import pallas as pl
from jax.experimental.pallas import tpu as pltpu

_B = 32
_C = 64
_P = 128
_L = 16
_COLS = _P
_ROWS = _B * _C * _L
_BLOCK_ROWS = 8192


def _mask_kernel(x_ref, moe_ref, meo_ref, oe_ref, eo_ref):
    x = x_ref[...]
    shape = x.shape
    row = jax.lax.broadcasted_iota(jnp.int32, shape, 0)
    col = jax.lax.broadcasted_iota(jnp.int32, shape, 1)
    oe = ((row // _L) ^ col) & 1
    eo = oe ^ 1
    oe_ref[...] = oe
    eo_ref[...] = eo
    zero = jnp.zeros_like(x)
    moe_ref[...] = jnp.where(oe == 1, zero, x)
    meo_ref[...] = jnp.where(oe == 0, zero, x)


def kernel(inputs):
    x2d = jnp.transpose(inputs, (0, 1, 3, 2)).reshape(_ROWS, _COLS)
    grid = (_ROWS // _BLOCK_ROWS,)
    spec = pl.BlockSpec((_BLOCK_ROWS, _COLS), lambda i: (i, 0))
    out = pl.pallas_call(
        _mask_kernel,
        grid=grid,
        in_specs=[spec],
        out_specs=[spec, spec, spec, spec],
        out_shape=[
            jax.ShapeDtypeStruct((_ROWS, _COLS), jnp.float32),
            jax.ShapeDtypeStruct((_ROWS, _COLS), jnp.float32),
            jax.ShapeDtypeStruct((_ROWS, _COLS), jnp.int32),
            jax.ShapeDtypeStruct((_ROWS, _COLS), jnp.int32),
        ],
        compiler_params=pltpu.CompilerParams(
            dimension_semantics=("parallel",),
        ),
    )(x2d)

    def _back(a):
        return jnp.transpose(
            a.reshape(_B, _C, _L, _P), (0, 1, 3, 2)
        )

    return tuple(_back(a) for a in out)
